# baseline (speedup 1.0000x reference)
import functools

import jax
import jax.numpy as jnp
from jax import lax
from jax.experimental import pallas as pl
from jax.experimental.pallas import tpu as pltpu
from jax.experimental.pallas import tpu_sc as plsc

_NC = 2
_NS = 16
_LANES = 16
_NW = _NC * _NS


def _sc_gather_build(B, C):
    bpw = B // _NW
    mesh = plsc.VectorSubcoreMesh(core_axis_name="c", subcore_axis_name="s")

    @functools.partial(
        pl.kernel,
        mesh=mesh,
        out_type=[
            jax.ShapeDtypeStruct((B,), jnp.float32),
            jax.ShapeDtypeStruct((B,), jnp.float32),
        ],
        scratch_types=[
            pltpu.VMEM((bpw,), jnp.int32),
            pltpu.VMEM((bpw,), jnp.int32),
            pltpu.VMEM((bpw,), jnp.float32),
            pltpu.VMEM((bpw,), jnp.float32),
            pltpu.SemaphoreType.DMA,
        ],
    )
    def sc_gather(ct_hbm, ctm_hbm, tgt_hbm, ctt_out, ctmt_out,
                  tgt_v, idx_v, a_v, b_v, sem):
        wid = lax.axis_index("s") * _NC + lax.axis_index("c")
        base = wid * bpw
        pltpu.sync_copy(tgt_hbm.at[pl.ds(base, bpw)], tgt_v)
        for k in range(bpw // _LANES):
            t = tgt_v[pl.ds(k * _LANES, _LANES)]
            rows = lax.iota(jnp.int32, _LANES) + (k * _LANES)
            idx_v[pl.ds(k * _LANES, _LANES)] = (rows + base) * C + t
        pltpu.async_copy(ct_hbm.at[idx_v], a_v, sem).wait()
        pltpu.async_copy(ctm_hbm.at[idx_v], b_v, sem).wait()
        pltpu.sync_copy(a_v, ctt_out.at[pl.ds(base, bpw)])
        pltpu.sync_copy(b_v, ctmt_out.at[pl.ds(base, bpw)])

    return sc_gather


def _main_build(B, C, W):
    nb = (C + W - 1) // W

    def body(ct_ref, tgt_ref, ctt_ref, ctmt_ref, xn_ref,
             oh_ref, loss_ref, lossg_ref, acc_ref):
        j = pl.program_id(0)
        ids = (j * W) + lax.broadcasted_iota(jnp.int32, (B, W), 1)
        x = ct_ref[...]
        e = jnp.exp(jnp.where(ids < C, x, -1e30))
        part = jnp.sum(e, axis=1, keepdims=True)

        @pl.when(j == 0)
        def _():
            acc_ref[...] = part

        @pl.when(j > 0)
        def _():
            acc_ref[...] += part

        oh_ref[...] = jnp.where(ids == tgt_ref[...], 1.0, 0.0).astype(jnp.float32)

        @pl.when(j == nb - 1)
        def _():
            ctmt = ctmt_ref[...]
            z = acc_ref[...] - jnp.exp(ctt_ref[...]) + jnp.exp(ctmt)
            li = jnp.log(z) - ctmt
            loss_ref[...] = jnp.sum(li, axis=0, keepdims=True) * (1.0 / B)
            xn = xn_ref[...]
            g = (1.0 / (110.0 * 110.0)) * xn + 1.0 / xn
            lossg_ref[...] = jnp.sum(g, axis=0, keepdims=True) * (1.0 / B)

    small = pl.BlockSpec((B, 1), lambda j: (0, 0))
    return pl.pallas_call(
        body,
        grid=(nb,),
        in_specs=[pl.BlockSpec((B, W), lambda j: (0, j)),
                  small, small, small, small],
        out_specs=[pl.BlockSpec((B, W), lambda j: (0, j)),
                   pl.BlockSpec((1, 1), lambda j: (0, 0)),
                   pl.BlockSpec((1, 1), lambda j: (0, 0))],
        out_shape=[jax.ShapeDtypeStruct((B, C), jnp.float32),
                   jax.ShapeDtypeStruct((1, 1), jnp.float32),
                   jax.ShapeDtypeStruct((1, 1), jnp.float32)],
        scratch_shapes=[pltpu.VMEM((B, 1), jnp.float32)],
        compiler_params=pltpu.CompilerParams(
            dimension_semantics=("arbitrary",)),
    )


def kernel(cos_theta, cos_theta_m, x_norm, target):
    B, C = cos_theta.shape
    tgt = target.astype(jnp.int32)
    ctt, ctmt = _sc_gather_build(B, C)(
        cos_theta.reshape(-1), cos_theta_m.reshape(-1), tgt)
    one_hot, loss, loss_g = _main_build(B, C, 2048)(
        cos_theta, tgt.reshape(B, 1), ctt.reshape(B, 1),
        ctmt.reshape(B, 1), x_norm)
    return (loss[0, 0], loss_g[0, 0], one_hot)


# speedup vs baseline: 1.7412x; 1.7412x over previous
import functools

import jax
import jax.numpy as jnp
from jax import lax
from jax.experimental import pallas as pl
from jax.experimental.pallas import tpu as pltpu
from jax.experimental.pallas import tpu_sc as plsc

_NC = 2
_NS = 16
_LANES = 16
_NW = _NC * _NS


def _sc_gather_build(B, C):
    bpw = B // _NW
    mesh = plsc.VectorSubcoreMesh(core_axis_name="c", subcore_axis_name="s")

    @functools.partial(
        pl.kernel,
        mesh=mesh,
        out_type=jax.ShapeDtypeStruct((B, _LANES), jnp.float32),
        scratch_types=[
            pltpu.VMEM((bpw,), jnp.int32),
            pltpu.VMEM((bpw, _LANES), jnp.float32),
            pltpu.SemaphoreType.DMA,
        ],
        compiler_params=pltpu.CompilerParams(use_tc_tiling_on_sc=True),
    )
    def sc_gather(tab_hbm, tgt_hbm, out_hbm, tgt_v, win_v, sem):
        wid = lax.axis_index("s") * _NC + lax.axis_index("c")
        base = wid * bpw
        pltpu.sync_copy(tgt_hbm.at[pl.ds(base, bpw)], tgt_v)
        copies = []
        for r in range(bpw):
            chunk, lane = divmod(r, _LANES)
            tvec = tgt_v[pl.ds(chunk * _LANES, _LANES)]
            col = (tvec[lane] // _LANES) * _LANES
            copies.append(pltpu.async_copy(
                tab_hbm.at[base + r, pl.ds(col, _LANES)],
                win_v.at[r], sem))
        for cp in copies:
            cp.wait()
        pltpu.sync_copy(win_v, out_hbm.at[pl.ds(base, bpw)])

    return sc_gather


def _main_build(B, C, W):
    nb = (C + W - 1) // W

    def body(ct_ref, tgt_ref, win_ref, xn_ref,
             oh_ref, loss_ref, lossg_ref, acc_ref, ctt_ref):
        j = pl.program_id(0)
        ids = (j * W) + lax.broadcasted_iota(jnp.int32, (B, W), 1)
        x = ct_ref[...]
        eq = ids == tgt_ref[...]
        e = jnp.exp(jnp.where(ids < C, x, -1e30))
        part = jnp.sum(e, axis=1, keepdims=True)
        ctp = jnp.sum(jnp.where(eq, x, 0.0), axis=1, keepdims=True)

        @pl.when(j == 0)
        def _():
            acc_ref[...] = part
            ctt_ref[...] = ctp

        @pl.when(j > 0)
        def _():
            acc_ref[...] += part
            ctt_ref[...] += ctp

        oh_ref[...] = jnp.where(eq, 1.0, 0.0).astype(jnp.float32)

        @pl.when(j == nb - 1)
        def _():
            tgt = tgt_ref[...]
            lane = tgt - (tgt // _LANES) * _LANES
            li16 = lax.broadcasted_iota(jnp.int32, (B, _LANES), 1)
            ctmt = jnp.sum(
                jnp.where(li16 == lane, win_ref[...], 0.0),
                axis=1, keepdims=True)
            z = acc_ref[...] - jnp.exp(ctt_ref[...]) + jnp.exp(ctmt)
            li = jnp.log(z) - ctmt
            loss_ref[...] = jnp.sum(li, axis=0, keepdims=True) * (1.0 / B)
            xn = xn_ref[...]
            g = (1.0 / (110.0 * 110.0)) * xn + 1.0 / xn
            lossg_ref[...] = jnp.sum(g, axis=0, keepdims=True) * (1.0 / B)

    small = pl.BlockSpec((B, 1), lambda j: (0, 0))
    return pl.pallas_call(
        body,
        grid=(nb,),
        in_specs=[pl.BlockSpec((B, W), lambda j: (0, j)),
                  small,
                  pl.BlockSpec((B, _LANES), lambda j: (0, 0)),
                  small],
        out_specs=[pl.BlockSpec((B, W), lambda j: (0, j)),
                   pl.BlockSpec((1, 1), lambda j: (0, 0)),
                   pl.BlockSpec((1, 1), lambda j: (0, 0))],
        out_shape=[jax.ShapeDtypeStruct((B, C), jnp.float32),
                   jax.ShapeDtypeStruct((1, 1), jnp.float32),
                   jax.ShapeDtypeStruct((1, 1), jnp.float32)],
        scratch_shapes=[pltpu.VMEM((B, 1), jnp.float32),
                        pltpu.VMEM((B, 1), jnp.float32)],
        compiler_params=pltpu.CompilerParams(
            dimension_semantics=("arbitrary",)),
    )


def kernel(cos_theta, cos_theta_m, x_norm, target):
    B, C = cos_theta.shape
    tgt = target.astype(jnp.int32)
    win = _sc_gather_build(B, C)(cos_theta_m, tgt)
    one_hot, loss, loss_g = _main_build(B, C, 2048)(
        cos_theta, tgt.reshape(B, 1), win, x_norm)
    return (loss[0, 0], loss_g[0, 0], one_hot)
